# baseline (speedup 1.0000x reference)
import functools

import jax
import jax.numpy as jnp
from jax import lax
from jax.experimental import pallas as pl
from jax.experimental.pallas import tpu as pltpu
from jax.experimental.pallas import tpu_sc as plsc

_NMS_T = 0.5
_SCORE_T = 0.01
_MAXDET = 100
_NEG_INF = float("-inf")

_NSL = 4
_KC = 512
_KLO = 256
_KHI = 512


def _score_kernel(cls_ref, s_ref, l_ref):
    x = cls_ref[0]
    c = x.shape[0]
    m = jnp.max(x, axis=0)
    ci = lax.broadcasted_iota(jnp.int32, x.shape, 0)
    lab = jnp.min(jnp.where(x == m[None, :], ci, c), axis=0)
    s_ref[0, 0, :] = m
    l_ref[0, 0, :] = lab


def _scores_labels(classification):
    B, N, C = classification.shape
    cls_t = jnp.transpose(classification, (0, 2, 1))
    s_flat, l_flat = pl.pallas_call(
        _score_kernel,
        grid=(B,),
        in_specs=[pl.BlockSpec((1, C, N), lambda b: (b, 0, 0))],
        out_specs=[
            pl.BlockSpec((1, 1, N), lambda b: (b, 0, 0)),
            pl.BlockSpec((1, 1, N), lambda b: (b, 0, 0)),
        ],
        out_shape=[
            jax.ShapeDtypeStruct((B, 1, N), jnp.float32),
            jax.ShapeDtypeStruct((B, 1, N), jnp.int32),
        ],
    )(cls_t)
    return s_flat.reshape(B, N), l_flat.reshape(B, N)


def _bisect_kernel(s_ref, t_ref, code_ref):
    B, N = s_ref.shape
    s = s_ref[:, :]
    cnt0 = jnp.sum((s > _SCORE_T).astype(jnp.int32), axis=1, keepdims=True)
    complete = cnt0 <= _KHI

    def body(i, carry):
        lo, hi, t, found_i = carry
        found = found_i > 0
        mid = (lo + hi) * 0.5
        cnt = jnp.sum((s > mid).astype(jnp.int32), axis=1, keepdims=True)
        inwin = (cnt >= _KLO) & (cnt <= _KHI)
        t = jnp.where(inwin & (~found), mid, t)
        lo = jnp.where((~found) & (cnt > _KHI), mid, lo)
        hi = jnp.where((~found) & (cnt < _KLO), mid, hi)
        return lo, hi, t, jnp.maximum(found_i, inwin.astype(jnp.int32))

    init = (
        jnp.full((B, 1), _SCORE_T, jnp.float32),
        jnp.full((B, 1), 1.0, jnp.float32),
        jnp.full((B, 1), _SCORE_T, jnp.float32),
        complete.astype(jnp.int32),
    )
    _, _, t, found_i = lax.fori_loop(0, 30, body, init)
    found = found_i > 0
    code = jnp.where(complete, 2, jnp.where(found, 1, 0))
    t_ref[:, :] = jnp.broadcast_to(t, (B, 16))
    code_ref[:, :] = jnp.broadcast_to(code, (B, 16))


def _bisect(scores):
    B, N = scores.shape
    return pl.pallas_call(
        _bisect_kernel,
        out_shape=[
            jax.ShapeDtypeStruct((B, 16), jnp.float32),
            jax.ShapeDtypeStruct((B, 16), jnp.int32),
        ],
    )(scores)


def _sc_compact_kernel(s_hbm, x1_hbm, y1_hbm, x2_hbm, y2_hbm, lab_hbm, t_hbm,
                       os_hbm, ox1_hbm, oy1_hbm, ox2_hbm, oy2_hbm, olab_hbm,
                       sbuf, posbuf, gsrc, shidx, gidx, gidx2, cs, cx1, cy1,
                       cx2, cy2, clab, tv, sem):
    n_total = s_hbm.shape[0] - 8
    B = t_hbm.shape[0]
    N = n_total // B
    slice_len = N // _NSL
    spad = sbuf.shape[0]
    nch = spad // 16
    nrow = spad // 128

    cid = lax.axis_index("c")
    sid = lax.axis_index("s")
    wid = sid * 2 + cid
    b = wid // _NSL
    sl = wid % _NSL
    base = b * N + sl * slice_len
    reg = _KC + 16
    region = wid * reg

    pltpu.sync_copy(s_hbm.at[pl.ds(base, slice_len)], sbuf.at[pl.ds(0, slice_len)])
    pltpu.sync_copy(t_hbm.at[b], tv)

    sent16 = jnp.full((16,), n_total, jnp.int32)
    for k2 in range(reg // 16):
        gidx[pl.ds(k2 * 16, 16)] = sent16
    pltpu.sync_copy(gidx, shidx.at[pl.ds(region, reg)])

    lane = lax.iota(jnp.int32, 16)
    tvec = tv[...]

    _gdn = lax.GatherDimensionNumbers(
        offset_dims=(), collapsed_slice_dims=(0,), start_index_map=(0,))

    def lanegather(x, idx):
        return lax.gather(x, idx[:, None], _gdn, (1,),
                          mode=lax.GatherScatterMode.PROMISE_IN_BOUNDS)

    def chunk(i, cnt):
        off = i * 16
        offv = jnp.full((16,), off, jnp.int32)
        sv = sbuf[pl.ds(off, 16)]
        m = (sv > tvec) & ((offv + lane) < slice_len)
        pref = jnp.where(m, 1, 0)
        for k in (1, 2, 4, 8):
            shifted = lanegather(pref, jnp.maximum(lane - k, 0))
            pref = pref + jnp.where(lane >= k, shifted, 0)
        pos = cnt + pref - 1
        m = m & (pos < _KC)
        tpos = jnp.full((16,), region, jnp.int32) + jnp.where(
            m, pos, jnp.full((16,), _KC, jnp.int32))
        posbuf[0, pl.ds(off, 16)] = tpos
        gsrc[pl.ds(off, 16)] = jnp.full((16,), base + off, jnp.int32) + lane
        return cnt + lanegather(pref, jnp.full((16,), 15, jnp.int32))

    lax.fori_loop(0, nch, chunk, jnp.zeros((16,), jnp.int32))

    pltpu.async_copy(gsrc, shidx.at[posbuf.at[0]], sem).wait()
    pltpu.sync_copy(shidx.at[pl.ds(region, _KC)], gidx.at[pl.ds(0, _KC)])

    for k2 in range(_KC // 16):
        gidx2[0, pl.ds(k2 * 16, 16)] = gidx[pl.ds(k2 * 16, 16)]

    descs = []
    for j in range(1):
        descs.append(pltpu.async_copy(s_hbm.at[gidx2.at[j]], cs, sem))
        descs.append(pltpu.async_copy(x1_hbm.at[gidx2.at[j]], cx1, sem))
        descs.append(pltpu.async_copy(y1_hbm.at[gidx2.at[j]], cy1, sem))
        descs.append(pltpu.async_copy(x2_hbm.at[gidx2.at[j]], cx2, sem))
        descs.append(pltpu.async_copy(y2_hbm.at[gidx2.at[j]], cy2, sem))
        descs.append(pltpu.async_copy(lab_hbm.at[gidx2.at[j]], clab, sem))
    for d in descs:
        d.wait()

    pltpu.sync_copy(cs, os_hbm.at[b, sl])
    pltpu.sync_copy(cx1, ox1_hbm.at[b, sl])
    pltpu.sync_copy(cy1, oy1_hbm.at[b, sl])
    pltpu.sync_copy(cx2, ox2_hbm.at[b, sl])
    pltpu.sync_copy(cy2, oy2_hbm.at[b, sl])
    pltpu.sync_copy(clab, olab_hbm.at[b, sl])


def _sc_compact(scores, x1, y1, x2, y2, labels, t16):
    B, N = scores.shape
    slice_len = N // _NSL
    spad = ((slice_len + 127) // 128) * 128
    f32 = jnp.float32
    out_pl = jax.ShapeDtypeStruct((B, _NSL, _KC), f32)
    out_i = jax.ShapeDtypeStruct((B, _NSL, _KC), jnp.int32)
    mesh = plsc.VectorSubcoreMesh(core_axis_name="c", subcore_axis_name="s")
    run = pl.kernel(
        _sc_compact_kernel,
        out_type=[out_pl, out_pl, out_pl, out_pl, out_pl, out_i],
        mesh=mesh,
        scratch_types=[
            pltpu.VMEM((spad,), f32),
            pltpu.VMEM((1, spad), jnp.int32),
            pltpu.VMEM((spad,), jnp.int32),
            pltpu.VMEM_SHARED((32 * (_KC + 16),), jnp.int32),
            pltpu.VMEM((_KC + 16,), jnp.int32),
            pltpu.VMEM((1, _KC), jnp.int32),
            pltpu.VMEM((_KC,), f32),
            pltpu.VMEM((_KC,), f32),
            pltpu.VMEM((_KC,), f32),
            pltpu.VMEM((_KC,), f32),
            pltpu.VMEM((_KC,), f32),
            pltpu.VMEM((_KC,), jnp.int32),
            pltpu.VMEM((16,), f32),
            pltpu.SemaphoreType.DMA,
        ],
    )
    neg8 = jnp.full((8,), _NEG_INF, jnp.float32)
    zero8 = jnp.zeros((8,), jnp.float32)
    zi8 = jnp.zeros((8,), jnp.int32)
    cs, cx1, cy1, cx2, cy2, clab = run(
        jnp.concatenate([scores.reshape(B * N), neg8]),
        jnp.concatenate([x1.reshape(B * N), zero8]),
        jnp.concatenate([y1.reshape(B * N), zero8]),
        jnp.concatenate([x2.reshape(B * N), zero8]),
        jnp.concatenate([y2.reshape(B * N), zero8]),
        jnp.concatenate([labels.reshape(B * N), zi8]), t16,
    )
    cw = _NSL * _KC
    return (cs.reshape(B, cw), cx1.reshape(B, cw), cy1.reshape(B, cw),
            cx2.reshape(B, cw), cy2.reshape(B, cw), clab.reshape(B, cw))


def _greedy_pass(n, iters, get_planes, out_refs, oiota):
    cur_ref, x1_ref, y1_ref, x2_ref, y2_ref, lab_ref, a2_ref = get_planes
    os_ref, ox1_ref, oy1_ref, ox2_ref, oy2_ref, ol_ref = out_refs
    B = os_ref.shape[0]
    iota = lax.broadcasted_iota(jnp.int32, (B, n), 1)

    def step(i, kept):
        cur = cur_ref[:, :]
        m = jnp.max(cur, axis=1, keepdims=True)
        hit = cur == m
        idx = jnp.min(jnp.where(hit, iota, n), axis=1, keepdims=True)
        one = iota == idx

        X1 = x1_ref[:, :]
        Y1 = y1_ref[:, :]
        X2 = x2_ref[:, :]
        Y2 = y2_ref[:, :]
        bx1 = jnp.sum(jnp.where(one, X1, 0.0), axis=1, keepdims=True)
        by1 = jnp.sum(jnp.where(one, Y1, 0.0), axis=1, keepdims=True)
        bx2 = jnp.sum(jnp.where(one, X2, 0.0), axis=1, keepdims=True)
        by2 = jnp.sum(jnp.where(one, Y2, 0.0), axis=1, keepdims=True)
        blab = jnp.sum(jnp.where(one, lab_ref[:, :], 0), axis=1, keepdims=True)

        xx1 = jnp.maximum(bx1, X1)
        yy1 = jnp.maximum(by1, Y1)
        xx2 = jnp.minimum(bx2, X2)
        yy2 = jnp.minimum(by2, Y2)
        inter = jnp.maximum(xx2 - xx1, 0.0) * jnp.maximum(yy2 - yy1, 0.0)
        a1 = (bx2 - bx1) * (by2 - by1)
        iou = inter / (a1 + a2_ref[:, :] - inter + 1e-8)
        sup = iou > _NMS_T
        cur_ref[:, :] = jnp.where(sup | one, _NEG_INF, cur)

        valid = m > _NEG_INF
        sel = oiota == i
        os_ref[:, :] = jnp.where(sel, jnp.where(valid, m, -1.0), os_ref[:, :])
        ox1_ref[:, :] = jnp.where(sel, jnp.where(valid, bx1, -1.0), ox1_ref[:, :])
        oy1_ref[:, :] = jnp.where(sel, jnp.where(valid, by1, -1.0), oy1_ref[:, :])
        ox2_ref[:, :] = jnp.where(sel, jnp.where(valid, bx2, -1.0), ox2_ref[:, :])
        oy2_ref[:, :] = jnp.where(sel, jnp.where(valid, by2, -1.0), oy2_ref[:, :])
        ol_ref[:, :] = jnp.where(sel, jnp.where(valid, blab, -1), ol_ref[:, :])
        return kept + valid.astype(jnp.int32)

    return lax.fori_loop(0, iters, step, jnp.zeros((B, 1), jnp.int32))


def _greedy_kernel(cs_ref, cx1_ref, cy1_ref, cx2_ref, cy2_ref, clab_ref,
                   code_ref, s_ref, x1_ref, y1_ref, x2_ref, y2_ref, lab_ref,
                   os_ref, ox1_ref, oy1_ref, ox2_ref, oy2_ref, ol_ref,
                   curc_ref, a2c_ref, cur_ref, a2_ref):
    B, CW = cs_ref.shape
    N = s_ref.shape[1]
    oiota = lax.broadcasted_iota(jnp.int32, (B, _MAXDET), 1)
    out_refs = (os_ref, ox1_ref, oy1_ref, ox2_ref, oy2_ref, ol_ref)

    curc_ref[:, :] = cs_ref[:, :]
    a2c_ref[:, :] = ((cx2_ref[:, :] - cx1_ref[:, :])
                     * (cy2_ref[:, :] - cy1_ref[:, :]))
    kept = _greedy_pass(
        CW, _MAXDET,
        (curc_ref, cx1_ref, cy1_ref, cx2_ref, cy2_ref, clab_ref, a2c_ref),
        out_refs, oiota)

    code = code_ref[:, 0:1]
    need_fb = (code == 0) | ((code == 1) & (kept < _MAXDET))
    any_fb = jnp.max(need_fb.astype(jnp.int32))

    def fallback():
        s = s_ref[:, :]
        cur_ref[:, :] = jnp.where(s > _SCORE_T, s, _NEG_INF)
        a2_ref[:, :] = ((x2_ref[:, :] - x1_ref[:, :])
                        * (y2_ref[:, :] - y1_ref[:, :]))
        _greedy_pass(
            N, _MAXDET,
            (cur_ref, x1_ref, y1_ref, x2_ref, y2_ref, lab_ref, a2_ref),
            out_refs, oiota)

    lax.cond(any_fb > 0, fallback, lambda: None)


def kernel(boxes, classification):
    B, N, C = classification.shape
    scores, labels = _scores_labels(classification)
    x1 = boxes[..., 0]
    y1 = boxes[..., 1]
    x2 = boxes[..., 2]
    y2 = boxes[..., 3]

    t16, code16 = _bisect(scores)
    cs, cx1, cy1, cx2, cy2, clab = _sc_compact(scores, x1, y1, x2, y2, labels, t16)

    outs = pl.pallas_call(
        _greedy_kernel,
        out_shape=[
            jax.ShapeDtypeStruct((B, _MAXDET), jnp.float32),
            jax.ShapeDtypeStruct((B, _MAXDET), jnp.float32),
            jax.ShapeDtypeStruct((B, _MAXDET), jnp.float32),
            jax.ShapeDtypeStruct((B, _MAXDET), jnp.float32),
            jax.ShapeDtypeStruct((B, _MAXDET), jnp.float32),
            jax.ShapeDtypeStruct((B, _MAXDET), jnp.int32),
        ],
        scratch_shapes=[
            pltpu.VMEM((B, _NSL * _KC), jnp.float32),
            pltpu.VMEM((B, _NSL * _KC), jnp.float32),
            pltpu.VMEM((B, N), jnp.float32),
            pltpu.VMEM((B, N), jnp.float32),
        ],
    )(cs, cx1, cy1, cx2, cy2, clab, code16, scores, x1, y1, x2, y2, labels)
    os, ox1, oy1, ox2, oy2, ol = outs
    out_boxes = jnp.stack([ox1, oy1, ox2, oy2], axis=-1)
    return (out_boxes, os, ol)


# speedup vs baseline: 1.0120x; 1.0120x over previous
import functools

import jax
import jax.numpy as jnp
from jax import lax
from jax.experimental import pallas as pl
from jax.experimental.pallas import tpu as pltpu
from jax.experimental.pallas import tpu_sc as plsc

_NMS_T = 0.5
_SCORE_T = 0.01
_MAXDET = 100
_NEG_INF = float("-inf")

_NSL = 4
_KC = 512
_KLO = 256
_KHI = 512


def _score_kernel(cls_ref, s_ref, l_ref):
    x = cls_ref[0]
    c = x.shape[0]
    m = jnp.max(x, axis=0)
    ci = lax.broadcasted_iota(jnp.int32, x.shape, 0)
    lab = jnp.min(jnp.where(x == m[None, :], ci, c), axis=0)
    s_ref[0, 0, :] = m
    l_ref[0, 0, :] = lab


def _scores_labels(classification):
    B, N, C = classification.shape
    cls_t = jnp.transpose(classification, (0, 2, 1))
    s_flat, l_flat = pl.pallas_call(
        _score_kernel,
        grid=(B,),
        in_specs=[pl.BlockSpec((1, C, N), lambda b: (b, 0, 0))],
        out_specs=[
            pl.BlockSpec((1, 1, N), lambda b: (b, 0, 0)),
            pl.BlockSpec((1, 1, N), lambda b: (b, 0, 0)),
        ],
        out_shape=[
            jax.ShapeDtypeStruct((B, 1, N), jnp.float32),
            jax.ShapeDtypeStruct((B, 1, N), jnp.int32),
        ],
    )(cls_t)
    return s_flat.reshape(B, N), l_flat.reshape(B, N)


def _bisect_kernel(s_ref, t_ref, code_ref):
    B, N = s_ref.shape
    s = s_ref[:, :]
    cnt0 = jnp.sum((s > _SCORE_T).astype(jnp.int32), axis=1, keepdims=True)
    complete = cnt0 <= _KHI

    def body(i, carry):
        lo, hi, t, found_i = carry
        found = found_i > 0
        mid = (lo + hi) * 0.5
        cnt = jnp.sum((s > mid).astype(jnp.int32), axis=1, keepdims=True)
        inwin = (cnt >= _KLO) & (cnt <= _KHI)
        t = jnp.where(inwin & (~found), mid, t)
        lo = jnp.where((~found) & (cnt > _KHI), mid, lo)
        hi = jnp.where((~found) & (cnt < _KLO), mid, hi)
        return lo, hi, t, jnp.maximum(found_i, inwin.astype(jnp.int32))

    init = (
        jnp.full((B, 1), _SCORE_T, jnp.float32),
        jnp.full((B, 1), 1.0, jnp.float32),
        jnp.full((B, 1), _SCORE_T, jnp.float32),
        complete.astype(jnp.int32),
    )
    _, _, t, found_i = lax.fori_loop(0, 30, body, init)
    found = found_i > 0
    code = jnp.where(complete, 2, jnp.where(found, 1, 0))
    t_ref[:, :] = jnp.broadcast_to(t, (B, 16))
    code_ref[:, :] = jnp.broadcast_to(code, (B, 16))


def _bisect(scores):
    B, N = scores.shape
    return pl.pallas_call(
        _bisect_kernel,
        out_shape=[
            jax.ShapeDtypeStruct((B, 16), jnp.float32),
            jax.ShapeDtypeStruct((B, 16), jnp.int32),
        ],
    )(scores)


def _sc_compact_kernel(s_hbm, x1_hbm, y1_hbm, x2_hbm, y2_hbm, lab_hbm, t_hbm,
                       os_hbm, ox1_hbm, oy1_hbm, ox2_hbm, oy2_hbm, olab_hbm,
                       sbuf, posbuf, gsrc, shidx, gidx, gidx2, cs, cx1, cy1,
                       cx2, cy2, clab, tv, sem):
    n_total = s_hbm.shape[0] - 8
    B = t_hbm.shape[0]
    N = n_total // B
    slice_len = N // _NSL
    spad = sbuf.shape[0]
    nch = spad // 16
    nrow = spad // 128

    cid = lax.axis_index("c")
    sid = lax.axis_index("s")
    wid = sid * 2 + cid
    b = wid // _NSL
    sl = wid % _NSL
    base = b * N + sl * slice_len
    reg = _KC + 16
    region = wid * reg

    pltpu.sync_copy(s_hbm.at[pl.ds(base, slice_len)], sbuf.at[pl.ds(0, slice_len)])
    pltpu.sync_copy(t_hbm.at[b], tv)

    sent16 = jnp.full((16,), n_total, jnp.int32)
    for k2 in range(reg // 16):
        gidx[pl.ds(k2 * 16, 16)] = sent16
    pltpu.sync_copy(gidx, shidx.at[pl.ds(region, reg)])

    lane = lax.iota(jnp.int32, 16)
    tvec = tv[...]

    _gdn = lax.GatherDimensionNumbers(
        offset_dims=(), collapsed_slice_dims=(0,), start_index_map=(0,))

    def lanegather(x, idx):
        return lax.gather(x, idx[:, None], _gdn, (1,),
                          mode=lax.GatherScatterMode.PROMISE_IN_BOUNDS)

    def chunk(i, cnt):
        off = i * 16
        offv = jnp.full((16,), off, jnp.int32)
        sv = sbuf[pl.ds(off, 16)]
        m = (sv > tvec) & ((offv + lane) < slice_len)
        pref = jnp.where(m, 1, 0)
        for k in (1, 2, 4, 8):
            shifted = lanegather(pref, jnp.maximum(lane - k, 0))
            pref = pref + jnp.where(lane >= k, shifted, 0)
        pos = cnt + pref - 1
        m = m & (pos < _KC)
        tpos = jnp.full((16,), region, jnp.int32) + jnp.where(
            m, pos, jnp.full((16,), _KC, jnp.int32))
        r = off // 128
        c0 = off % 128
        posbuf[r, pl.ds(c0, 16)] = tpos
        gsrc[pl.ds(off, 16)] = jnp.full((16,), base + off, jnp.int32) + lane
        return cnt + lanegather(pref, jnp.full((16,), 15, jnp.int32))

    lax.fori_loop(0, nch, chunk, jnp.zeros((16,), jnp.int32))

    descs = []
    for j in range(nrow):
        descs.append(pltpu.async_copy(
            gsrc.at[pl.ds(j * 128, 128)], shidx.at[posbuf.at[j]], sem))
    for d in descs:
        d.wait()
    pltpu.sync_copy(shidx.at[pl.ds(region, _KC)], gidx.at[pl.ds(0, _KC)])

    for k2 in range(_KC // 16):
        gidx2[0, pl.ds(k2 * 16, 16)] = gidx[pl.ds(k2 * 16, 16)]

    descs = []
    for j in range(1):
        descs.append(pltpu.async_copy(s_hbm.at[gidx2.at[j]], cs, sem))
        descs.append(pltpu.async_copy(x1_hbm.at[gidx2.at[j]], cx1, sem))
        descs.append(pltpu.async_copy(y1_hbm.at[gidx2.at[j]], cy1, sem))
        descs.append(pltpu.async_copy(x2_hbm.at[gidx2.at[j]], cx2, sem))
        descs.append(pltpu.async_copy(y2_hbm.at[gidx2.at[j]], cy2, sem))
        descs.append(pltpu.async_copy(lab_hbm.at[gidx2.at[j]], clab, sem))
    for d in descs:
        d.wait()

    pltpu.sync_copy(cs, os_hbm.at[b, sl])
    pltpu.sync_copy(cx1, ox1_hbm.at[b, sl])
    pltpu.sync_copy(cy1, oy1_hbm.at[b, sl])
    pltpu.sync_copy(cx2, ox2_hbm.at[b, sl])
    pltpu.sync_copy(cy2, oy2_hbm.at[b, sl])
    pltpu.sync_copy(clab, olab_hbm.at[b, sl])


def _sc_compact(scores, x1, y1, x2, y2, labels, t16):
    B, N = scores.shape
    slice_len = N // _NSL
    spad = ((slice_len + 127) // 128) * 128
    f32 = jnp.float32
    out_pl = jax.ShapeDtypeStruct((B, _NSL, _KC), f32)
    out_i = jax.ShapeDtypeStruct((B, _NSL, _KC), jnp.int32)
    mesh = plsc.VectorSubcoreMesh(core_axis_name="c", subcore_axis_name="s")
    run = pl.kernel(
        _sc_compact_kernel,
        out_type=[out_pl, out_pl, out_pl, out_pl, out_pl, out_i],
        mesh=mesh,
        scratch_types=[
            pltpu.VMEM((spad,), f32),
            pltpu.VMEM((spad // 128, 128), jnp.int32),
            pltpu.VMEM((spad,), jnp.int32),
            pltpu.VMEM_SHARED((32 * (_KC + 16),), jnp.int32),
            pltpu.VMEM((_KC + 16,), jnp.int32),
            pltpu.VMEM((1, _KC), jnp.int32),
            pltpu.VMEM((_KC,), f32),
            pltpu.VMEM((_KC,), f32),
            pltpu.VMEM((_KC,), f32),
            pltpu.VMEM((_KC,), f32),
            pltpu.VMEM((_KC,), f32),
            pltpu.VMEM((_KC,), jnp.int32),
            pltpu.VMEM((16,), f32),
            pltpu.SemaphoreType.DMA,
        ],
    )
    neg8 = jnp.full((8,), _NEG_INF, jnp.float32)
    zero8 = jnp.zeros((8,), jnp.float32)
    zi8 = jnp.zeros((8,), jnp.int32)
    cs, cx1, cy1, cx2, cy2, clab = run(
        jnp.concatenate([scores.reshape(B * N), neg8]),
        jnp.concatenate([x1.reshape(B * N), zero8]),
        jnp.concatenate([y1.reshape(B * N), zero8]),
        jnp.concatenate([x2.reshape(B * N), zero8]),
        jnp.concatenate([y2.reshape(B * N), zero8]),
        jnp.concatenate([labels.reshape(B * N), zi8]), t16,
    )
    cw = _NSL * _KC
    return (cs.reshape(B, cw), cx1.reshape(B, cw), cy1.reshape(B, cw),
            cx2.reshape(B, cw), cy2.reshape(B, cw), clab.reshape(B, cw))


def _greedy_pass(n, iters, get_planes, out_refs, oiota):
    cur_ref, x1_ref, y1_ref, x2_ref, y2_ref, lab_ref, a2_ref = get_planes
    os_ref, ox1_ref, oy1_ref, ox2_ref, oy2_ref, ol_ref = out_refs
    B = os_ref.shape[0]
    iota = lax.broadcasted_iota(jnp.int32, (B, n), 1)

    def step(i, kept):
        cur = cur_ref[:, :]
        m = jnp.max(cur, axis=1, keepdims=True)
        hit = cur == m
        idx = jnp.min(jnp.where(hit, iota, n), axis=1, keepdims=True)
        one = iota == idx

        X1 = x1_ref[:, :]
        Y1 = y1_ref[:, :]
        X2 = x2_ref[:, :]
        Y2 = y2_ref[:, :]
        bx1 = jnp.sum(jnp.where(one, X1, 0.0), axis=1, keepdims=True)
        by1 = jnp.sum(jnp.where(one, Y1, 0.0), axis=1, keepdims=True)
        bx2 = jnp.sum(jnp.where(one, X2, 0.0), axis=1, keepdims=True)
        by2 = jnp.sum(jnp.where(one, Y2, 0.0), axis=1, keepdims=True)
        blab = jnp.sum(jnp.where(one, lab_ref[:, :], 0), axis=1, keepdims=True)

        xx1 = jnp.maximum(bx1, X1)
        yy1 = jnp.maximum(by1, Y1)
        xx2 = jnp.minimum(bx2, X2)
        yy2 = jnp.minimum(by2, Y2)
        inter = jnp.maximum(xx2 - xx1, 0.0) * jnp.maximum(yy2 - yy1, 0.0)
        a1 = (bx2 - bx1) * (by2 - by1)
        iou = inter / (a1 + a2_ref[:, :] - inter + 1e-8)
        sup = iou > _NMS_T
        cur_ref[:, :] = jnp.where(sup | one, _NEG_INF, cur)

        valid = m > _NEG_INF
        sel = oiota == i
        os_ref[:, :] = jnp.where(sel, jnp.where(valid, m, -1.0), os_ref[:, :])
        ox1_ref[:, :] = jnp.where(sel, jnp.where(valid, bx1, -1.0), ox1_ref[:, :])
        oy1_ref[:, :] = jnp.where(sel, jnp.where(valid, by1, -1.0), oy1_ref[:, :])
        ox2_ref[:, :] = jnp.where(sel, jnp.where(valid, bx2, -1.0), ox2_ref[:, :])
        oy2_ref[:, :] = jnp.where(sel, jnp.where(valid, by2, -1.0), oy2_ref[:, :])
        ol_ref[:, :] = jnp.where(sel, jnp.where(valid, blab, -1), ol_ref[:, :])
        return kept + valid.astype(jnp.int32)

    return lax.fori_loop(0, iters, step, jnp.zeros((B, 1), jnp.int32))


def _greedy_kernel(cs_ref, cx1_ref, cy1_ref, cx2_ref, cy2_ref, clab_ref,
                   code_ref, s_ref, x1_ref, y1_ref, x2_ref, y2_ref, lab_ref,
                   os_ref, ox1_ref, oy1_ref, ox2_ref, oy2_ref, ol_ref,
                   curc_ref, a2c_ref, cur_ref, a2_ref):
    B, CW = cs_ref.shape
    N = s_ref.shape[1]
    oiota = lax.broadcasted_iota(jnp.int32, (B, _MAXDET), 1)
    out_refs = (os_ref, ox1_ref, oy1_ref, ox2_ref, oy2_ref, ol_ref)

    curc_ref[:, :] = cs_ref[:, :]
    a2c_ref[:, :] = ((cx2_ref[:, :] - cx1_ref[:, :])
                     * (cy2_ref[:, :] - cy1_ref[:, :]))
    kept = _greedy_pass(
        CW, _MAXDET,
        (curc_ref, cx1_ref, cy1_ref, cx2_ref, cy2_ref, clab_ref, a2c_ref),
        out_refs, oiota)

    code = code_ref[:, 0:1]
    need_fb = (code == 0) | ((code == 1) & (kept < _MAXDET))
    any_fb = jnp.max(need_fb.astype(jnp.int32))

    def fallback():
        s = s_ref[:, :]
        cur_ref[:, :] = jnp.where(s > _SCORE_T, s, _NEG_INF)
        a2_ref[:, :] = ((x2_ref[:, :] - x1_ref[:, :])
                        * (y2_ref[:, :] - y1_ref[:, :]))
        _greedy_pass(
            N, _MAXDET,
            (cur_ref, x1_ref, y1_ref, x2_ref, y2_ref, lab_ref, a2_ref),
            out_refs, oiota)

    lax.cond(any_fb > 0, fallback, lambda: None)


def kernel(boxes, classification):
    B, N, C = classification.shape
    scores, labels = _scores_labels(classification)
    x1 = boxes[..., 0]
    y1 = boxes[..., 1]
    x2 = boxes[..., 2]
    y2 = boxes[..., 3]

    t16, code16 = _bisect(scores)
    cs, cx1, cy1, cx2, cy2, clab = _sc_compact(scores, x1, y1, x2, y2, labels, t16)

    outs = pl.pallas_call(
        _greedy_kernel,
        out_shape=[
            jax.ShapeDtypeStruct((B, _MAXDET), jnp.float32),
            jax.ShapeDtypeStruct((B, _MAXDET), jnp.float32),
            jax.ShapeDtypeStruct((B, _MAXDET), jnp.float32),
            jax.ShapeDtypeStruct((B, _MAXDET), jnp.float32),
            jax.ShapeDtypeStruct((B, _MAXDET), jnp.float32),
            jax.ShapeDtypeStruct((B, _MAXDET), jnp.int32),
        ],
        scratch_shapes=[
            pltpu.VMEM((B, _NSL * _KC), jnp.float32),
            pltpu.VMEM((B, _NSL * _KC), jnp.float32),
            pltpu.VMEM((B, N), jnp.float32),
            pltpu.VMEM((B, N), jnp.float32),
        ],
    )(cs, cx1, cy1, cx2, cy2, clab, code16, scores, x1, y1, x2, y2, labels)
    os, ox1, oy1, ox2, oy2, ol = outs
    out_boxes = jnp.stack([ox1, oy1, ox2, oy2], axis=-1)
    return (out_boxes, os, ol)


# speedup vs baseline: 1.5829x; 1.5642x over previous
import functools

import jax
import jax.numpy as jnp
from jax import lax
from jax.experimental import pallas as pl
from jax.experimental.pallas import tpu as pltpu
from jax.experimental.pallas import tpu_sc as plsc

_NMS_T = 0.5
_SCORE_T = 0.01
_MAXDET = 100
_NEG_INF = float("-inf")

_NSL = 4
_KC = 512
_KLO = 256
_KHI = 512


def _score_kernel(cls_ref, s_ref, l_ref):
    x = cls_ref[0]
    c = x.shape[0]
    m = jnp.max(x, axis=0)
    ci = lax.broadcasted_iota(jnp.int32, x.shape, 0)
    lab = jnp.min(jnp.where(x == m[None, :], ci, c), axis=0)
    s_ref[0, 0, :] = m
    l_ref[0, 0, :] = lab


def _scores_labels(classification):
    B, N, C = classification.shape
    cls_t = jnp.transpose(classification, (0, 2, 1))
    s_flat, l_flat = pl.pallas_call(
        _score_kernel,
        grid=(B,),
        in_specs=[pl.BlockSpec((1, C, N), lambda b: (b, 0, 0))],
        out_specs=[
            pl.BlockSpec((1, 1, N), lambda b: (b, 0, 0)),
            pl.BlockSpec((1, 1, N), lambda b: (b, 0, 0)),
        ],
        out_shape=[
            jax.ShapeDtypeStruct((B, 1, N), jnp.float32),
            jax.ShapeDtypeStruct((B, 1, N), jnp.int32),
        ],
    )(cls_t)
    return s_flat.reshape(B, N), l_flat.reshape(B, N)


def _bisect_kernel(s_ref, t_ref, code_ref):
    B, N = s_ref.shape
    s = s_ref[:, :]
    cnt0 = jnp.sum((s > _SCORE_T).astype(jnp.int32), axis=1, keepdims=True)
    complete = cnt0 <= _KHI

    def body(i, carry):
        lo, hi, t, found_i = carry
        found = found_i > 0
        mid = (lo + hi) * 0.5
        cnt = jnp.sum((s > mid).astype(jnp.int32), axis=1, keepdims=True)
        inwin = (cnt >= _KLO) & (cnt <= _KHI)
        t = jnp.where(inwin & (~found), mid, t)
        lo = jnp.where((~found) & (cnt > _KHI), mid, lo)
        hi = jnp.where((~found) & (cnt < _KLO), mid, hi)
        return lo, hi, t, jnp.maximum(found_i, inwin.astype(jnp.int32))

    init = (
        jnp.full((B, 1), _SCORE_T, jnp.float32),
        jnp.full((B, 1), 1.0, jnp.float32),
        jnp.full((B, 1), _SCORE_T, jnp.float32),
        complete.astype(jnp.int32),
    )
    _, _, t, found_i = lax.fori_loop(0, 30, body, init)
    found = found_i > 0
    code = jnp.where(complete, 2, jnp.where(found, 1, 0))
    t_ref[:, :] = jnp.broadcast_to(t, (B, 16))
    code_ref[:, :] = jnp.broadcast_to(code, (B, 16))


def _bisect(scores):
    B, N = scores.shape
    return pl.pallas_call(
        _bisect_kernel,
        out_shape=[
            jax.ShapeDtypeStruct((B, 16), jnp.float32),
            jax.ShapeDtypeStruct((B, 16), jnp.int32),
        ],
    )(scores)


def _sc_compact_kernel(s_hbm, x1_hbm, y1_hbm, x2_hbm, y2_hbm, lab_hbm, t_hbm,
                       os_hbm, ox1_hbm, oy1_hbm, ox2_hbm, oy2_hbm, olab_hbm,
                       sbuf, posbuf, gsrc, shidx, gidx, gidx2, cs, cx1, cy1,
                       cx2, cy2, clab, tv, sem):
    n_total = s_hbm.shape[0] - 8
    B = t_hbm.shape[0]
    N = n_total // B
    slice_len = N // _NSL
    spad = sbuf.shape[0]
    nch = spad // 16
    nrow = spad // 128

    cid = lax.axis_index("c")
    sid = lax.axis_index("s")
    wid = sid * 2 + cid
    b = wid // _NSL
    sl = wid % _NSL
    base = b * N + sl * slice_len
    reg = _KC + 16
    region = wid * reg

    pltpu.sync_copy(s_hbm.at[pl.ds(base, slice_len)], sbuf.at[pl.ds(0, slice_len)])
    pltpu.sync_copy(t_hbm.at[b], tv)

    sent16 = jnp.full((16,), n_total, jnp.int32)
    for k2 in range(reg // 16):
        gidx[pl.ds(k2 * 16, 16)] = sent16
    pltpu.sync_copy(gidx, shidx.at[pl.ds(region, reg)])

    lane = lax.iota(jnp.int32, 16)
    tvec = tv[...]

    _gdn = lax.GatherDimensionNumbers(
        offset_dims=(), collapsed_slice_dims=(0,), start_index_map=(0,))

    def lanegather(x, idx):
        return lax.gather(x, idx[:, None], _gdn, (1,),
                          mode=lax.GatherScatterMode.PROMISE_IN_BOUNDS)

    def chunk(i, cnt):
        off = i * 16
        offv = jnp.full((16,), off, jnp.int32)
        sv = sbuf[pl.ds(off, 16)]
        m = (sv > tvec) & ((offv + lane) < slice_len)
        pref = jnp.where(m, 1, 0)
        for k in (1, 2, 4, 8):
            shifted = lanegather(pref, jnp.maximum(lane - k, 0))
            pref = pref + jnp.where(lane >= k, shifted, 0)
        pos = cnt + pref - 1
        m = m & (pos < _KC)
        tpos = jnp.full((16,), region, jnp.int32) + jnp.where(
            m, pos, jnp.full((16,), _KC, jnp.int32))
        r = off // 128
        c0 = off % 128
        posbuf[r, pl.ds(c0, 16)] = tpos
        gsrc[pl.ds(off, 16)] = jnp.full((16,), base + off, jnp.int32) + lane
        return cnt + lanegather(pref, jnp.full((16,), 15, jnp.int32))

    lax.fori_loop(0, nch, chunk, jnp.zeros((16,), jnp.int32))

    descs = []
    for j in range(nrow):
        descs.append(pltpu.async_copy(
            gsrc.at[pl.ds(j * 128, 128)], shidx.at[posbuf.at[j]], sem))
    for d in descs:
        d.wait()
    pltpu.sync_copy(shidx.at[pl.ds(region, _KC)], gidx.at[pl.ds(0, _KC)])

    for j in range(_KC // 128):
        for k2 in range(8):
            gidx2[j, pl.ds(k2 * 16, 16)] = gidx[pl.ds(j * 128 + k2 * 16, 16)]

    descs = []
    for j in range(_KC // 128):
        isl = pl.ds(j * 128, 128)
        descs.append(pltpu.async_copy(s_hbm.at[gidx2.at[j]], cs.at[isl], sem))
        descs.append(pltpu.async_copy(x1_hbm.at[gidx2.at[j]], cx1.at[isl], sem))
        descs.append(pltpu.async_copy(y1_hbm.at[gidx2.at[j]], cy1.at[isl], sem))
        descs.append(pltpu.async_copy(x2_hbm.at[gidx2.at[j]], cx2.at[isl], sem))
        descs.append(pltpu.async_copy(y2_hbm.at[gidx2.at[j]], cy2.at[isl], sem))
        descs.append(pltpu.async_copy(lab_hbm.at[gidx2.at[j]], clab.at[isl], sem))
    for d in descs:
        d.wait()

    pltpu.sync_copy(cs, os_hbm.at[b, sl])
    pltpu.sync_copy(cx1, ox1_hbm.at[b, sl])
    pltpu.sync_copy(cy1, oy1_hbm.at[b, sl])
    pltpu.sync_copy(cx2, ox2_hbm.at[b, sl])
    pltpu.sync_copy(cy2, oy2_hbm.at[b, sl])
    pltpu.sync_copy(clab, olab_hbm.at[b, sl])


def _sc_compact(scores, x1, y1, x2, y2, labels, t16):
    B, N = scores.shape
    slice_len = N // _NSL
    spad = ((slice_len + 127) // 128) * 128
    f32 = jnp.float32
    out_pl = jax.ShapeDtypeStruct((B, _NSL, _KC), f32)
    out_i = jax.ShapeDtypeStruct((B, _NSL, _KC), jnp.int32)
    mesh = plsc.VectorSubcoreMesh(core_axis_name="c", subcore_axis_name="s")
    run = pl.kernel(
        _sc_compact_kernel,
        out_type=[out_pl, out_pl, out_pl, out_pl, out_pl, out_i],
        mesh=mesh,
        scratch_types=[
            pltpu.VMEM((spad,), f32),
            pltpu.VMEM((spad // 128, 128), jnp.int32),
            pltpu.VMEM((spad,), jnp.int32),
            pltpu.VMEM_SHARED((32 * (_KC + 16),), jnp.int32),
            pltpu.VMEM((_KC + 16,), jnp.int32),
            pltpu.VMEM((_KC // 128, 128), jnp.int32),
            pltpu.VMEM((_KC,), f32),
            pltpu.VMEM((_KC,), f32),
            pltpu.VMEM((_KC,), f32),
            pltpu.VMEM((_KC,), f32),
            pltpu.VMEM((_KC,), f32),
            pltpu.VMEM((_KC,), jnp.int32),
            pltpu.VMEM((16,), f32),
            pltpu.SemaphoreType.DMA,
        ],
    )
    neg8 = jnp.full((8,), _NEG_INF, jnp.float32)
    zero8 = jnp.zeros((8,), jnp.float32)
    zi8 = jnp.zeros((8,), jnp.int32)
    cs, cx1, cy1, cx2, cy2, clab = run(
        jnp.concatenate([scores.reshape(B * N), neg8]),
        jnp.concatenate([x1.reshape(B * N), zero8]),
        jnp.concatenate([y1.reshape(B * N), zero8]),
        jnp.concatenate([x2.reshape(B * N), zero8]),
        jnp.concatenate([y2.reshape(B * N), zero8]),
        jnp.concatenate([labels.reshape(B * N), zi8]), t16,
    )
    cw = _NSL * _KC
    return (cs.reshape(B, cw), cx1.reshape(B, cw), cy1.reshape(B, cw),
            cx2.reshape(B, cw), cy2.reshape(B, cw), clab.reshape(B, cw))


def _greedy_pass(n, iters, get_planes, out_refs, oiota):
    cur_ref, x1_ref, y1_ref, x2_ref, y2_ref, lab_ref, a2_ref = get_planes
    os_ref, ox1_ref, oy1_ref, ox2_ref, oy2_ref, ol_ref = out_refs
    B = os_ref.shape[0]
    iota = lax.broadcasted_iota(jnp.int32, (B, n), 1)

    def step(i, kept):
        cur = cur_ref[:, :]
        m = jnp.max(cur, axis=1, keepdims=True)
        hit = cur == m
        idx = jnp.min(jnp.where(hit, iota, n), axis=1, keepdims=True)
        one = iota == idx

        X1 = x1_ref[:, :]
        Y1 = y1_ref[:, :]
        X2 = x2_ref[:, :]
        Y2 = y2_ref[:, :]
        bx1 = jnp.sum(jnp.where(one, X1, 0.0), axis=1, keepdims=True)
        by1 = jnp.sum(jnp.where(one, Y1, 0.0), axis=1, keepdims=True)
        bx2 = jnp.sum(jnp.where(one, X2, 0.0), axis=1, keepdims=True)
        by2 = jnp.sum(jnp.where(one, Y2, 0.0), axis=1, keepdims=True)
        blab = jnp.sum(jnp.where(one, lab_ref[:, :], 0), axis=1, keepdims=True)

        xx1 = jnp.maximum(bx1, X1)
        yy1 = jnp.maximum(by1, Y1)
        xx2 = jnp.minimum(bx2, X2)
        yy2 = jnp.minimum(by2, Y2)
        inter = jnp.maximum(xx2 - xx1, 0.0) * jnp.maximum(yy2 - yy1, 0.0)
        a1 = (bx2 - bx1) * (by2 - by1)
        iou = inter / (a1 + a2_ref[:, :] - inter + 1e-8)
        sup = iou > _NMS_T
        cur_ref[:, :] = jnp.where(sup | one, _NEG_INF, cur)

        valid = m > _NEG_INF
        sel = oiota == i
        os_ref[:, :] = jnp.where(sel, jnp.where(valid, m, -1.0), os_ref[:, :])
        ox1_ref[:, :] = jnp.where(sel, jnp.where(valid, bx1, -1.0), ox1_ref[:, :])
        oy1_ref[:, :] = jnp.where(sel, jnp.where(valid, by1, -1.0), oy1_ref[:, :])
        ox2_ref[:, :] = jnp.where(sel, jnp.where(valid, bx2, -1.0), ox2_ref[:, :])
        oy2_ref[:, :] = jnp.where(sel, jnp.where(valid, by2, -1.0), oy2_ref[:, :])
        ol_ref[:, :] = jnp.where(sel, jnp.where(valid, blab, -1), ol_ref[:, :])
        return kept + valid.astype(jnp.int32)

    return lax.fori_loop(0, iters, step, jnp.zeros((B, 1), jnp.int32))


def _greedy_kernel(cs_ref, cx1_ref, cy1_ref, cx2_ref, cy2_ref, clab_ref,
                   code_ref, s_ref, x1_ref, y1_ref, x2_ref, y2_ref, lab_ref,
                   os_ref, ox1_ref, oy1_ref, ox2_ref, oy2_ref, ol_ref,
                   curc_ref, a2c_ref, cur_ref, a2_ref):
    B, CW = cs_ref.shape
    N = s_ref.shape[1]
    oiota = lax.broadcasted_iota(jnp.int32, (B, _MAXDET), 1)
    out_refs = (os_ref, ox1_ref, oy1_ref, ox2_ref, oy2_ref, ol_ref)

    curc_ref[:, :] = cs_ref[:, :]
    a2c_ref[:, :] = ((cx2_ref[:, :] - cx1_ref[:, :])
                     * (cy2_ref[:, :] - cy1_ref[:, :]))
    kept = _greedy_pass(
        CW, _MAXDET,
        (curc_ref, cx1_ref, cy1_ref, cx2_ref, cy2_ref, clab_ref, a2c_ref),
        out_refs, oiota)

    code = code_ref[:, 0:1]
    need_fb = (code == 0) | ((code == 1) & (kept < _MAXDET))
    any_fb = jnp.max(need_fb.astype(jnp.int32))

    def fallback():
        s = s_ref[:, :]
        cur_ref[:, :] = jnp.where(s > _SCORE_T, s, _NEG_INF)
        a2_ref[:, :] = ((x2_ref[:, :] - x1_ref[:, :])
                        * (y2_ref[:, :] - y1_ref[:, :]))
        _greedy_pass(
            N, _MAXDET,
            (cur_ref, x1_ref, y1_ref, x2_ref, y2_ref, lab_ref, a2_ref),
            out_refs, oiota)

    lax.cond(any_fb > 0, fallback, lambda: None)


def kernel(boxes, classification):
    B, N, C = classification.shape
    scores, labels = _scores_labels(classification)
    x1 = boxes[..., 0]
    y1 = boxes[..., 1]
    x2 = boxes[..., 2]
    y2 = boxes[..., 3]

    t16, code16 = _bisect(scores)
    cs, cx1, cy1, cx2, cy2, clab = _sc_compact(scores, x1, y1, x2, y2, labels, t16)

    outs = pl.pallas_call(
        _greedy_kernel,
        out_shape=[
            jax.ShapeDtypeStruct((B, _MAXDET), jnp.float32),
            jax.ShapeDtypeStruct((B, _MAXDET), jnp.float32),
            jax.ShapeDtypeStruct((B, _MAXDET), jnp.float32),
            jax.ShapeDtypeStruct((B, _MAXDET), jnp.float32),
            jax.ShapeDtypeStruct((B, _MAXDET), jnp.float32),
            jax.ShapeDtypeStruct((B, _MAXDET), jnp.int32),
        ],
        scratch_shapes=[
            pltpu.VMEM((B, _NSL * _KC), jnp.float32),
            pltpu.VMEM((B, _NSL * _KC), jnp.float32),
            pltpu.VMEM((B, N), jnp.float32),
            pltpu.VMEM((B, N), jnp.float32),
        ],
    )(cs, cx1, cy1, cx2, cy2, clab, code16, scores, x1, y1, x2, y2, labels)
    os, ox1, oy1, ox2, oy2, ol = outs
    out_boxes = jnp.stack([ox1, oy1, ox2, oy2], axis=-1)
    return (out_boxes, os, ol)


# speedup vs baseline: 2.1473x; 1.3565x over previous
import functools

import jax
import jax.numpy as jnp
from jax import lax
from jax.experimental import pallas as pl
from jax.experimental.pallas import tpu as pltpu
from jax.experimental.pallas import tpu_sc as plsc

_NMS_T = 0.5
_SCORE_T = 0.01
_MAXDET = 100
_NEG_INF = float("-inf")

_NSL = 4
_KC = 256
_KLO = 256
_KHI = 512


def _score_kernel(cls_ref, s_ref, l_ref):
    x = cls_ref[0]
    c = x.shape[0]
    m = jnp.max(x, axis=0)
    ci = lax.broadcasted_iota(jnp.int32, x.shape, 0)
    lab = jnp.min(jnp.where(x == m[None, :], ci, c), axis=0)
    s_ref[0, 0, :] = m
    l_ref[0, 0, :] = lab


def _scores_labels(classification):
    B, N, C = classification.shape
    cls_t = jnp.transpose(classification, (0, 2, 1))
    s_flat, l_flat = pl.pallas_call(
        _score_kernel,
        grid=(B,),
        in_specs=[pl.BlockSpec((1, C, N), lambda b: (b, 0, 0))],
        out_specs=[
            pl.BlockSpec((1, 1, N), lambda b: (b, 0, 0)),
            pl.BlockSpec((1, 1, N), lambda b: (b, 0, 0)),
        ],
        out_shape=[
            jax.ShapeDtypeStruct((B, 1, N), jnp.float32),
            jax.ShapeDtypeStruct((B, 1, N), jnp.int32),
        ],
    )(cls_t)
    return s_flat.reshape(B, N), l_flat.reshape(B, N)


def _bisect_kernel(s_ref, t_ref, code_ref):
    B, N = s_ref.shape
    s = s_ref[:, :]
    cnt0 = jnp.sum((s > _SCORE_T).astype(jnp.int32), axis=1, keepdims=True)
    complete = cnt0 <= _KHI

    def body(i, carry):
        lo, hi, t, found_i = carry
        found = found_i > 0
        mid = (lo + hi) * 0.5
        cnt = jnp.sum((s > mid).astype(jnp.int32), axis=1, keepdims=True)
        inwin = (cnt >= _KLO) & (cnt <= _KHI)
        t = jnp.where(inwin & (~found), mid, t)
        lo = jnp.where((~found) & (cnt > _KHI), mid, lo)
        hi = jnp.where((~found) & (cnt < _KLO), mid, hi)
        return lo, hi, t, jnp.maximum(found_i, inwin.astype(jnp.int32))

    init = (
        jnp.full((B, 1), _SCORE_T, jnp.float32),
        jnp.full((B, 1), 1.0, jnp.float32),
        jnp.full((B, 1), _SCORE_T, jnp.float32),
        complete.astype(jnp.int32),
    )
    _, _, t, found_i = lax.fori_loop(0, 30, body, init)
    found = found_i > 0
    code = jnp.where(complete, 2, jnp.where(found, 1, 0))
    t_ref[:, :] = jnp.broadcast_to(t, (B, 16))
    code_ref[:, :] = jnp.broadcast_to(code, (B, 16))


def _bisect(scores):
    B, N = scores.shape
    return pl.pallas_call(
        _bisect_kernel,
        out_shape=[
            jax.ShapeDtypeStruct((B, 16), jnp.float32),
            jax.ShapeDtypeStruct((B, 16), jnp.int32),
        ],
    )(scores)


def _sc_compact_kernel(s_hbm, x1_hbm, y1_hbm, x2_hbm, y2_hbm, lab_hbm, t_hbm,
                       os_hbm, ox1_hbm, oy1_hbm, ox2_hbm, oy2_hbm, olab_hbm,
                       oflag_hbm,
                       sbuf, posbuf, gsrc, shidx, gidx, gidx2, cs, cx1, cy1,
                       cx2, cy2, clab, tv, fvec, sem):
    n_total = s_hbm.shape[0] - 8
    B = t_hbm.shape[0]
    N = n_total // B
    slice_len = N // _NSL
    spad = sbuf.shape[0]
    nch = spad // 16
    nrow = spad // 128

    cid = lax.axis_index("c")
    sid = lax.axis_index("s")
    wid = sid * 2 + cid
    b = wid // _NSL
    sl = wid % _NSL
    base = b * N + sl * slice_len
    reg = _KC + 16
    region = wid * reg

    pltpu.sync_copy(s_hbm.at[pl.ds(base, slice_len)], sbuf.at[pl.ds(0, slice_len)])
    pltpu.sync_copy(t_hbm.at[b], tv)

    sent16 = jnp.full((16,), n_total, jnp.int32)
    for k2 in range(reg // 16):
        gidx[pl.ds(k2 * 16, 16)] = sent16
    pltpu.sync_copy(gidx, shidx.at[pl.ds(region, reg)])

    lane = lax.iota(jnp.int32, 16)
    tvec = tv[...]

    _gdn = lax.GatherDimensionNumbers(
        offset_dims=(), collapsed_slice_dims=(0,), start_index_map=(0,))

    def lanegather(x, idx):
        return lax.gather(x, idx[:, None], _gdn, (1,),
                          mode=lax.GatherScatterMode.PROMISE_IN_BOUNDS)

    def chunk(i, cnt):
        off = i * 16
        offv = jnp.full((16,), off, jnp.int32)
        sv = sbuf[pl.ds(off, 16)]
        m = (sv > tvec) & ((offv + lane) < slice_len)
        pref = jnp.where(m, 1, 0)
        for k in (1, 2, 4, 8):
            shifted = lanegather(pref, jnp.maximum(lane - k, 0))
            pref = pref + jnp.where(lane >= k, shifted, 0)
        pos = cnt + pref - 1
        m = m & (pos < _KC)
        tpos = jnp.full((16,), region, jnp.int32) + jnp.where(
            m, pos, jnp.full((16,), _KC, jnp.int32))
        r = off // 128
        c0 = off % 128
        posbuf[r, pl.ds(c0, 16)] = tpos
        gsrc[pl.ds(off, 16)] = jnp.full((16,), base + off, jnp.int32) + lane
        return cnt + lanegather(pref, jnp.full((16,), 15, jnp.int32))

    cntf = lax.fori_loop(0, nch, chunk, jnp.zeros((16,), jnp.int32))
    fvec[...] = jnp.where(cntf > _KC, 1, 0)
    pltpu.sync_copy(fvec, oflag_hbm.at[wid])

    descs = []
    for j in range(nrow):
        descs.append(pltpu.async_copy(
            gsrc.at[pl.ds(j * 128, 128)], shidx.at[posbuf.at[j]], sem))
    for d in descs:
        d.wait()
    pltpu.sync_copy(shidx.at[pl.ds(region, _KC)], gidx.at[pl.ds(0, _KC)])

    for j in range(_KC // 128):
        for k2 in range(8):
            gidx2[j, pl.ds(k2 * 16, 16)] = gidx[pl.ds(j * 128 + k2 * 16, 16)]

    descs = []
    for j in range(_KC // 128):
        isl = pl.ds(j * 128, 128)
        descs.append(pltpu.async_copy(s_hbm.at[gidx2.at[j]], cs.at[isl], sem))
        descs.append(pltpu.async_copy(x1_hbm.at[gidx2.at[j]], cx1.at[isl], sem))
        descs.append(pltpu.async_copy(y1_hbm.at[gidx2.at[j]], cy1.at[isl], sem))
        descs.append(pltpu.async_copy(x2_hbm.at[gidx2.at[j]], cx2.at[isl], sem))
        descs.append(pltpu.async_copy(y2_hbm.at[gidx2.at[j]], cy2.at[isl], sem))
        descs.append(pltpu.async_copy(lab_hbm.at[gidx2.at[j]], clab.at[isl], sem))
    for d in descs:
        d.wait()

    pltpu.sync_copy(cs, os_hbm.at[b, sl])
    pltpu.sync_copy(cx1, ox1_hbm.at[b, sl])
    pltpu.sync_copy(cy1, oy1_hbm.at[b, sl])
    pltpu.sync_copy(cx2, ox2_hbm.at[b, sl])
    pltpu.sync_copy(cy2, oy2_hbm.at[b, sl])
    pltpu.sync_copy(clab, olab_hbm.at[b, sl])


def _sc_compact(scores, x1, y1, x2, y2, labels, t16):
    B, N = scores.shape
    slice_len = N // _NSL
    spad = ((slice_len + 127) // 128) * 128
    f32 = jnp.float32
    out_pl = jax.ShapeDtypeStruct((B, _NSL, _KC), f32)
    out_i = jax.ShapeDtypeStruct((B, _NSL, _KC), jnp.int32)
    mesh = plsc.VectorSubcoreMesh(core_axis_name="c", subcore_axis_name="s")
    run = pl.kernel(
        _sc_compact_kernel,
        out_type=[out_pl, out_pl, out_pl, out_pl, out_pl, out_i,
                  jax.ShapeDtypeStruct((B * _NSL, 16), jnp.int32)],
        mesh=mesh,
        scratch_types=[
            pltpu.VMEM((spad,), f32),
            pltpu.VMEM((spad // 128, 128), jnp.int32),
            pltpu.VMEM((spad,), jnp.int32),
            pltpu.VMEM_SHARED((32 * (_KC + 16),), jnp.int32),
            pltpu.VMEM((_KC + 16,), jnp.int32),
            pltpu.VMEM((_KC // 128, 128), jnp.int32),
            pltpu.VMEM((_KC,), f32),
            pltpu.VMEM((_KC,), f32),
            pltpu.VMEM((_KC,), f32),
            pltpu.VMEM((_KC,), f32),
            pltpu.VMEM((_KC,), f32),
            pltpu.VMEM((_KC,), jnp.int32),
            pltpu.VMEM((16,), f32),
            pltpu.VMEM((16,), jnp.int32),
            pltpu.SemaphoreType.DMA,
        ],
    )
    neg8 = jnp.full((8,), _NEG_INF, jnp.float32)
    zero8 = jnp.zeros((8,), jnp.float32)
    zi8 = jnp.zeros((8,), jnp.int32)
    cs, cx1, cy1, cx2, cy2, clab, oflag = run(
        jnp.concatenate([scores.reshape(B * N), neg8]),
        jnp.concatenate([x1.reshape(B * N), zero8]),
        jnp.concatenate([y1.reshape(B * N), zero8]),
        jnp.concatenate([x2.reshape(B * N), zero8]),
        jnp.concatenate([y2.reshape(B * N), zero8]),
        jnp.concatenate([labels.reshape(B * N), zi8]), t16,
    )
    cw = _NSL * _KC
    return (cs.reshape(B, cw), cx1.reshape(B, cw), cy1.reshape(B, cw),
            cx2.reshape(B, cw), cy2.reshape(B, cw), clab.reshape(B, cw),
            oflag.reshape(B, _NSL * 16))


def _greedy_pass(n, iters, get_planes, out_refs, oiota):
    cur_ref, x1_ref, y1_ref, x2_ref, y2_ref, lab_ref, a2_ref = get_planes
    os_ref, ox1_ref, oy1_ref, ox2_ref, oy2_ref, ol_ref = out_refs
    B = os_ref.shape[0]
    iota = lax.broadcasted_iota(jnp.int32, (B, n), 1)

    def step(i, kept):
        cur = cur_ref[:, :]
        m = jnp.max(cur, axis=1, keepdims=True)
        hit = cur == m
        idx = jnp.min(jnp.where(hit, iota, n), axis=1, keepdims=True)
        one = iota == idx

        X1 = x1_ref[:, :]
        Y1 = y1_ref[:, :]
        X2 = x2_ref[:, :]
        Y2 = y2_ref[:, :]
        bx1 = jnp.sum(jnp.where(one, X1, 0.0), axis=1, keepdims=True)
        by1 = jnp.sum(jnp.where(one, Y1, 0.0), axis=1, keepdims=True)
        bx2 = jnp.sum(jnp.where(one, X2, 0.0), axis=1, keepdims=True)
        by2 = jnp.sum(jnp.where(one, Y2, 0.0), axis=1, keepdims=True)
        blab = jnp.sum(jnp.where(one, lab_ref[:, :], 0), axis=1, keepdims=True)

        xx1 = jnp.maximum(bx1, X1)
        yy1 = jnp.maximum(by1, Y1)
        xx2 = jnp.minimum(bx2, X2)
        yy2 = jnp.minimum(by2, Y2)
        inter = jnp.maximum(xx2 - xx1, 0.0) * jnp.maximum(yy2 - yy1, 0.0)
        a1 = (bx2 - bx1) * (by2 - by1)
        iou = inter / (a1 + a2_ref[:, :] - inter + 1e-8)
        sup = iou > _NMS_T
        cur_ref[:, :] = jnp.where(sup | one, _NEG_INF, cur)

        valid = m > _NEG_INF
        sel = oiota == i
        os_ref[:, :] = jnp.where(sel, jnp.where(valid, m, -1.0), os_ref[:, :])
        ox1_ref[:, :] = jnp.where(sel, jnp.where(valid, bx1, -1.0), ox1_ref[:, :])
        oy1_ref[:, :] = jnp.where(sel, jnp.where(valid, by1, -1.0), oy1_ref[:, :])
        ox2_ref[:, :] = jnp.where(sel, jnp.where(valid, bx2, -1.0), ox2_ref[:, :])
        oy2_ref[:, :] = jnp.where(sel, jnp.where(valid, by2, -1.0), oy2_ref[:, :])
        ol_ref[:, :] = jnp.where(sel, jnp.where(valid, blab, -1), ol_ref[:, :])
        return kept + valid.astype(jnp.int32)

    return lax.fori_loop(0, iters, step, jnp.zeros((B, 1), jnp.int32))


def _greedy_kernel(cs_ref, cx1_ref, cy1_ref, cx2_ref, cy2_ref, clab_ref,
                   code_ref, flag_ref, s_ref, x1_ref, y1_ref, x2_ref, y2_ref,
                   lab_ref,
                   os_ref, ox1_ref, oy1_ref, ox2_ref, oy2_ref, ol_ref,
                   curc_ref, a2c_ref, cur_ref, a2_ref):
    B, CW = cs_ref.shape
    N = s_ref.shape[1]
    oiota = lax.broadcasted_iota(jnp.int32, (B, _MAXDET), 1)
    out_refs = (os_ref, ox1_ref, oy1_ref, ox2_ref, oy2_ref, ol_ref)

    curc_ref[:, :] = cs_ref[:, :]
    a2c_ref[:, :] = ((cx2_ref[:, :] - cx1_ref[:, :])
                     * (cy2_ref[:, :] - cy1_ref[:, :]))
    kept = _greedy_pass(
        CW, _MAXDET,
        (curc_ref, cx1_ref, cy1_ref, cx2_ref, cy2_ref, clab_ref, a2c_ref),
        out_refs, oiota)

    code = code_ref[:, 0:1]
    ovf = jnp.max(flag_ref[:, :], axis=1, keepdims=True) > 0
    need_fb = ovf | (code == 0) | ((code == 1) & (kept < _MAXDET))
    any_fb = jnp.max(need_fb.astype(jnp.int32))

    def fallback():
        s = s_ref[:, :]
        cur_ref[:, :] = jnp.where(s > _SCORE_T, s, _NEG_INF)
        a2_ref[:, :] = ((x2_ref[:, :] - x1_ref[:, :])
                        * (y2_ref[:, :] - y1_ref[:, :]))
        _greedy_pass(
            N, _MAXDET,
            (cur_ref, x1_ref, y1_ref, x2_ref, y2_ref, lab_ref, a2_ref),
            out_refs, oiota)

    lax.cond(any_fb > 0, fallback, lambda: None)


def kernel(boxes, classification):
    B, N, C = classification.shape
    scores, labels = _scores_labels(classification)
    x1 = boxes[..., 0]
    y1 = boxes[..., 1]
    x2 = boxes[..., 2]
    y2 = boxes[..., 3]

    t16, code16 = _bisect(scores)
    cs, cx1, cy1, cx2, cy2, clab, oflag = _sc_compact(
        scores, x1, y1, x2, y2, labels, t16)

    outs = pl.pallas_call(
        _greedy_kernel,
        out_shape=[
            jax.ShapeDtypeStruct((B, _MAXDET), jnp.float32),
            jax.ShapeDtypeStruct((B, _MAXDET), jnp.float32),
            jax.ShapeDtypeStruct((B, _MAXDET), jnp.float32),
            jax.ShapeDtypeStruct((B, _MAXDET), jnp.float32),
            jax.ShapeDtypeStruct((B, _MAXDET), jnp.float32),
            jax.ShapeDtypeStruct((B, _MAXDET), jnp.int32),
        ],
        scratch_shapes=[
            pltpu.VMEM((B, _NSL * _KC), jnp.float32),
            pltpu.VMEM((B, _NSL * _KC), jnp.float32),
            pltpu.VMEM((B, N), jnp.float32),
            pltpu.VMEM((B, N), jnp.float32),
        ],
    )(cs, cx1, cy1, cx2, cy2, clab, code16, oflag, scores, x1, y1, x2, y2,
      labels)
    os, ox1, oy1, ox2, oy2, ol = outs
    out_boxes = jnp.stack([ox1, oy1, ox2, oy2], axis=-1)
    return (out_boxes, os, ol)
